# baseline (speedup 1.0000x reference)
import jax
import jax.numpy as jnp
from jax.experimental import pallas as pl
from jax.experimental.pallas import tpu as pltpu

_N_USERS = 8192
_N_ITEMS = 8192
_EMB = 16
_N = _N_USERS + _N_ITEMS

_BM1 = 128
_NB1 = _N // _BM1
_BM2 = 256
_NB2 = _N // _BM2
_NSLOT = 4
_SUB = _BM2 // _BM1
_WR_CYCLE = _NSLOT * _SUB


def _dot_t(lhs, rhs_t):
    return jax.lax.dot_general(
        lhs, rhs_t, (((1,), (1,)), ((), ())),
        preferred_element_type=jnp.float32)


def _fused_body(a_hbm, e0t_ref, outt_ref, abf_hbm,
                fb, bb, e1b, eb, in_sems, wr_sems, rd_sems):
    eb[...] = e0t_ref[...]
    def start_in1(i):
        pltpu.make_async_copy(
            a_hbm.at[pl.ds(i * _BM1, _BM1), :],
            fb.at[jax.lax.rem(i, 2)],
            in_sems.at[jax.lax.rem(i, 2)],
        ).start()

    def wait_in1(i):
        pltpu.make_async_copy(
            a_hbm.at[pl.ds(i * _BM1, _BM1), :],
            fb.at[jax.lax.rem(i, 2)],
            in_sems.at[jax.lax.rem(i, 2)],
        ).wait()

    def wr_copy(i):
        slot = jax.lax.rem(jax.lax.div(i, _SUB), _NSLOT)
        sub = jax.lax.rem(i, _SUB)
        return pltpu.make_async_copy(
            bb.at[slot, pl.ds(sub * _BM1, _BM1), :],
            abf_hbm.at[pl.ds(i * _BM1, _BM1), :],
            wr_sems.at[jax.lax.rem(i, _WR_CYCLE)],
        )

    def start_rd2(j):
        pltpu.make_async_copy(
            abf_hbm.at[pl.ds(jax.lax.rem(j, _NB2) * _BM2, _BM2), :],
            bb.at[jax.lax.rem(j, _NSLOT)],
            rd_sems.at[jax.lax.rem(j, _NSLOT)],
        ).start()

    def wait_rd2(j):
        pltpu.make_async_copy(
            abf_hbm.at[pl.ds(jax.lax.rem(j, _NB2) * _BM2, _BM2), :],
            bb.at[jax.lax.rem(j, _NSLOT)],
            rd_sems.at[jax.lax.rem(j, _NSLOT)],
        ).wait()

    start_in1(0)
    start_in1(1)

    def phase1(i, _):
        wait_in1(i)
        islot = jax.lax.rem(i, 2)
        slot = jax.lax.rem(jax.lax.div(i, _SUB), _NSLOT)
        sub = jax.lax.rem(i, _SUB)

        @pl.when(i >= _WR_CYCLE)
        def _():
            wr_copy(i - _WR_CYCLE).wait()

        e1 = None
        nchunks = 4
        cw = _N // nchunks
        for c in range(nchunks):
            a_ch = fb[islot, :, pl.ds(c * cw, cw)]
            abf_ch = a_ch.astype(jnp.bfloat16)
            bb[slot, pl.ds(sub * _BM1, _BM1), pl.ds(c * cw, cw)] = abf_ch
            part = _dot_t(abf_ch, eb[:, pl.ds(c * cw, cw)])
            e1 = part if e1 is None else e1 + part
        wr_copy(i).start()

        rows = pl.ds(i * _BM1, _BM1)
        e1t = e1.T
        outt_ref[:, rows] = eb[:, rows].astype(jnp.float32) + e1t
        e1b[:, rows] = e1t.astype(jnp.bfloat16)

        @pl.when(i + 2 < _NB1)
        def _():
            start_in1(i + 2)

        return _

    jax.lax.fori_loop(0, _NB1, phase1, None)

    def drain(i, _):
        wr_copy(_NB1 - _WR_CYCLE + i).wait()
        return _

    jax.lax.fori_loop(0, _WR_CYCLE, drain, None)

    start_rd2(0)
    start_rd2(1)
    start_rd2(2)

    def phase23(j, _):
        wait_rd2(j)

        @pl.when(j + 3 < 2 * _NB2)
        def _():
            start_rd2(j + 3)

        a2 = bb[jax.lax.rem(j, _NSLOT)]
        rhs_t = jnp.where(j < _NB2, e1b[...], eb[...])
        e_next = _dot_t(a2, rhs_t)
        rows = pl.ds(jax.lax.rem(j, _NB2) * _BM2, _BM2)

        @pl.when(j < _NB2)
        def _():
            et = e_next.T
            eb[:, rows] = et.astype(jnp.bfloat16)
            outt_ref[:, rows] += et

        @pl.when(j >= _NB2)
        def _():
            outt_ref[:, rows] = 0.25 * (outt_ref[:, rows] + e_next.T)

        return _

    jax.lax.fori_loop(0, 2 * _NB2, phase23, None)


def kernel(norm_adj, user_emb, item_emb):
    e0t_bf = jnp.concatenate([user_emb, item_emb], axis=0).T.astype(jnp.bfloat16)

    outt, _ = pl.pallas_call(
        _fused_body,
        in_specs=[
            pl.BlockSpec(memory_space=pltpu.MemorySpace.HBM),
            pl.BlockSpec(memory_space=pltpu.MemorySpace.VMEM),
        ],
        out_specs=[
            pl.BlockSpec(memory_space=pltpu.MemorySpace.VMEM),
            pl.BlockSpec(memory_space=pltpu.MemorySpace.HBM),
        ],
        out_shape=[
            jax.ShapeDtypeStruct((_EMB, _N), jnp.float32),
            jax.ShapeDtypeStruct((_N, _N), jnp.bfloat16),
        ],
        scratch_shapes=[
            pltpu.VMEM((2, _BM1, _N), jnp.float32),
            pltpu.VMEM((_NSLOT, _BM2, _N), jnp.bfloat16),
            pltpu.VMEM((_EMB, _N), jnp.bfloat16),
            pltpu.VMEM((_EMB, _N), jnp.bfloat16),
            pltpu.SemaphoreType.DMA((2,)),
            pltpu.SemaphoreType.DMA((_WR_CYCLE,)),
            pltpu.SemaphoreType.DMA((_NSLOT,)),
        ],
        compiler_params=pltpu.CompilerParams(
            vmem_limit_bytes=100 * 1024 * 1024,
        ),
    )(norm_adj, e0t_bf)

    final = outt.T
    return (final[:_N_USERS], final[_N_USERS:])


# speedup vs baseline: 1.0059x; 1.0059x over previous
import jax
import jax.numpy as jnp
from jax.experimental import pallas as pl
from jax.experimental.pallas import tpu as pltpu

_N_USERS = 8192
_N_ITEMS = 8192
_EMB = 16
_N = _N_USERS + _N_ITEMS

_BM1 = 128
_NB1 = _N // _BM1
_BM2 = 256
_NB2 = _N // _BM2
_NSLOT = 3
_SUB = _BM2 // _BM1
_WR_CYCLE = _NSLOT * _SUB


def _dot_t(lhs, rhs_t):
    return jax.lax.dot_general(
        lhs, rhs_t, (((1,), (1,)), ((), ())),
        preferred_element_type=jnp.float32)


def _fused_body(a_hbm, e0t_ref, outt_ref, abf_hbm,
                fb, bb, e1b, eb, in_sems, wr_sems, rd_sems):
    eb[...] = e0t_ref[...]
    def start_in1(i):
        pltpu.make_async_copy(
            a_hbm.at[pl.ds(i * _BM1, _BM1), :],
            fb.at[jax.lax.rem(i, 3)],
            in_sems.at[jax.lax.rem(i, 3)],
        ).start()

    def wait_in1(i):
        pltpu.make_async_copy(
            a_hbm.at[pl.ds(i * _BM1, _BM1), :],
            fb.at[jax.lax.rem(i, 3)],
            in_sems.at[jax.lax.rem(i, 3)],
        ).wait()

    def wr_copy(i):
        slot = jax.lax.rem(jax.lax.div(i, _SUB), _NSLOT)
        sub = jax.lax.rem(i, _SUB)
        return pltpu.make_async_copy(
            bb.at[slot, pl.ds(sub * _BM1, _BM1), :],
            abf_hbm.at[pl.ds(i * _BM1, _BM1), :],
            wr_sems.at[jax.lax.rem(i, _WR_CYCLE)],
        )

    def start_rd2(j):
        pltpu.make_async_copy(
            abf_hbm.at[pl.ds(jax.lax.rem(j, _NB2) * _BM2, _BM2), :],
            bb.at[jax.lax.rem(j, _NSLOT)],
            rd_sems.at[jax.lax.rem(j, _NSLOT)],
        ).start()

    def wait_rd2(j):
        pltpu.make_async_copy(
            abf_hbm.at[pl.ds(jax.lax.rem(j, _NB2) * _BM2, _BM2), :],
            bb.at[jax.lax.rem(j, _NSLOT)],
            rd_sems.at[jax.lax.rem(j, _NSLOT)],
        ).wait()

    start_in1(0)
    start_in1(1)

    def phase1(i, _):
        wait_in1(i)

        @pl.when(i + 2 < _NB1)
        def _():
            start_in1(i + 2)

        islot = jax.lax.rem(i, 3)
        slot = jax.lax.rem(jax.lax.div(i, _SUB), _NSLOT)
        sub = jax.lax.rem(i, _SUB)

        @pl.when(i >= _WR_CYCLE)
        def _():
            wr_copy(i - _WR_CYCLE).wait()

        e1 = None
        nchunks = 4
        cw = _N // nchunks
        for c in range(nchunks):
            a_ch = fb[islot, :, pl.ds(c * cw, cw)]
            abf_ch = a_ch.astype(jnp.bfloat16)
            bb[slot, pl.ds(sub * _BM1, _BM1), pl.ds(c * cw, cw)] = abf_ch
            part = _dot_t(abf_ch, eb[:, pl.ds(c * cw, cw)])
            e1 = part if e1 is None else e1 + part
        wr_copy(i).start()

        rows = pl.ds(i * _BM1, _BM1)
        e1t = e1.T
        outt_ref[:, rows] = eb[:, rows].astype(jnp.float32) + e1t
        e1b[:, rows] = e1t.astype(jnp.bfloat16)

        return _

    jax.lax.fori_loop(0, _NB1, phase1, None)

    def drain(i, _):
        wr_copy(_NB1 - _WR_CYCLE + i).wait()
        return _

    jax.lax.fori_loop(0, _WR_CYCLE, drain, None)

    start_rd2(0)
    start_rd2(1)

    def phase2(j, _):
        wait_rd2(j)

        @pl.when(j + 2 < 2 * _NB2)
        def _():
            start_rd2(j + 2)

        a2 = bb[jax.lax.rem(j, _NSLOT)]
        e2 = _dot_t(a2, e1b[...])
        rows = pl.ds(j * _BM2, _BM2)
        e2t = e2.T
        eb[:, rows] = e2t.astype(jnp.bfloat16)
        outt_ref[:, rows] += e2t
        return _

    jax.lax.fori_loop(0, _NB2, phase2, None)

    def phase3(j, _):
        wait_rd2(j)

        @pl.when(j + 2 < 2 * _NB2)
        def _():
            start_rd2(j + 2)

        a2 = bb[jax.lax.rem(j, _NSLOT)]
        e3 = _dot_t(a2, eb[...])
        rows = pl.ds(jax.lax.rem(j, _NB2) * _BM2, _BM2)
        outt_ref[:, rows] = 0.25 * (outt_ref[:, rows] + e3.T)
        return _

    jax.lax.fori_loop(_NB2, 2 * _NB2, phase3, None)


def kernel(norm_adj, user_emb, item_emb):
    e0t_bf = jnp.concatenate([user_emb, item_emb], axis=0).T.astype(jnp.bfloat16)

    outt, _ = pl.pallas_call(
        _fused_body,
        in_specs=[
            pl.BlockSpec(memory_space=pltpu.MemorySpace.HBM),
            pl.BlockSpec(memory_space=pltpu.MemorySpace.VMEM),
        ],
        out_specs=[
            pl.BlockSpec(memory_space=pltpu.MemorySpace.VMEM),
            pl.BlockSpec(memory_space=pltpu.MemorySpace.HBM),
        ],
        out_shape=[
            jax.ShapeDtypeStruct((_EMB, _N), jnp.float32),
            jax.ShapeDtypeStruct((_N, _N), jnp.bfloat16),
        ],
        scratch_shapes=[
            pltpu.VMEM((3, _BM1, _N), jnp.float32),
            pltpu.VMEM((_NSLOT, _BM2, _N), jnp.bfloat16),
            pltpu.VMEM((_EMB, _N), jnp.bfloat16),
            pltpu.VMEM((_EMB, _N), jnp.bfloat16),
            pltpu.SemaphoreType.DMA((3,)),
            pltpu.SemaphoreType.DMA((_WR_CYCLE,)),
            pltpu.SemaphoreType.DMA((_NSLOT,)),
        ],
        compiler_params=pltpu.CompilerParams(
            vmem_limit_bytes=100 * 1024 * 1024,
        ),
    )(norm_adj, e0t_bf)

    final = outt.T
    return (final[:_N_USERS], final[_N_USERS:])
